# baseline (speedup 1.0000x reference)
import functools

import jax
import jax.numpy as jnp
from jax import lax
from jax.experimental import pallas as pl
from jax.experimental.pallas import tpu as pltpu
from jax.experimental.pallas import tpu_sc as plsc

_VOCAB = 100000
_BATCH = 4096
_SEQ = 200
_NUM_CORES = 2
_NUM_SUBCORES = 16
_NW = _NUM_CORES * _NUM_SUBCORES
_ROWS_W = _BATCH // _NW
_PER_W = _ROWS_W * _SEQ
_LANES = 16

_mesh = plsc.VectorSubcoreMesh(core_axis_name="c", subcore_axis_name="s")


@functools.partial(
    pl.kernel,
    mesh=_mesh,
    out_type=jax.ShapeDtypeStruct((_BATCH, _SEQ), jnp.int32),
    scratch_types=[
        pltpu.VMEM((_VOCAB,), jnp.int32),
        pltpu.VMEM((_ROWS_W, _SEQ), jnp.int32),
        pltpu.SemaphoreType.DMA,
        pltpu.SemaphoreType.DMA,
    ],
    compiler_params=pltpu.CompilerParams(
        needs_layout_passes=False, use_tc_tiling_on_sc=False
    ),
)
def _lookup(tokens_hbm, vocab_hbm, out_hbm, vocab_v, tok_v, sem_a, sem_b):
    wid = lax.axis_index("s") * _NUM_CORES + lax.axis_index("c")
    row0 = wid * _ROWS_W
    cp_vocab = pltpu.async_copy(vocab_hbm, vocab_v, sem_a)
    cp_tok = pltpu.async_copy(tokens_hbm.at[pl.ds(row0, _ROWS_W)], tok_v, sem_b)
    cp_vocab.wait()
    cp_tok.wait()

    @plsc.parallel_loop(0, _PER_W // _LANES, unroll=8)
    def _gather(i):
        p = i * _LANES + jnp.arange(_LANES, dtype=jnp.int32)
        r = p // _SEQ
        c = p - r * _SEQ
        toks = plsc.load_gather(tok_v, [r, c])
        plsc.store_scatter(tok_v, [r, c], plsc.load_gather(vocab_v, [toks]))

    pltpu.sync_copy(tok_v, out_hbm.at[pl.ds(row0, _ROWS_W)])


def kernel(tokens, vocab_map):
    return _lookup(tokens, vocab_map)


# speedup vs baseline: 1.0218x; 1.0218x over previous
import functools

import jax
import jax.numpy as jnp
from jax import lax
from jax.experimental import pallas as pl
from jax.experimental.pallas import tpu as pltpu
from jax.experimental.pallas import tpu_sc as plsc

_VOCAB = 100000
_BATCH = 4096
_SEQ = 200
_TOTAL = _BATCH * _SEQ
_NUM_CORES = 2
_NUM_SUBCORES = 16
_NW = _NUM_CORES * _NUM_SUBCORES
_PER_W = _TOTAL // _NW
_LANES = 16
_CHUNKS = 4
_CHUNK = _PER_W // _CHUNKS

_mesh = plsc.VectorSubcoreMesh(core_axis_name="c", subcore_axis_name="s")


@functools.partial(
    pl.kernel,
    mesh=_mesh,
    out_type=jax.ShapeDtypeStruct((_TOTAL,), jnp.int32),
    scratch_types=[
        pltpu.VMEM((_VOCAB,), jnp.int32),
        pltpu.VMEM((_PER_W,), jnp.int32),
        pltpu.SemaphoreType.DMA,
        pltpu.SemaphoreType.DMA,
        pltpu.SemaphoreType.DMA,
    ],
    compiler_params=pltpu.CompilerParams(needs_layout_passes=False),
)
def _lookup(tokens_hbm, vocab_hbm, out_hbm, vocab_v, tok_v, sem_a, sem_b, sem_o):
    wid = lax.axis_index("s") * _NUM_CORES + lax.axis_index("c")
    base = wid * _PER_W
    cp_vocab = pltpu.async_copy(vocab_hbm, vocab_v, sem_a)
    cp_tok = pltpu.async_copy(tokens_hbm.at[pl.ds(base, _PER_W)], tok_v, sem_b)
    cp_tok.wait()
    cp_vocab.wait()

    out_copies = []
    for c in range(_CHUNKS):
        lo = c * _CHUNK // _LANES
        hi = (c + 1) * _CHUNK // _LANES

        @plsc.parallel_loop(lo, hi, unroll=8)
        def _gather(i):
            idx = tok_v[pl.ds(i * _LANES, _LANES)]
            tok_v[pl.ds(i * _LANES, _LANES)] = plsc.load_gather(vocab_v, [idx])

        out_copies.append(
            pltpu.async_copy(
                tok_v.at[pl.ds(c * _CHUNK, _CHUNK)],
                out_hbm.at[pl.ds(base + c * _CHUNK, _CHUNK)],
                sem_o,
            )
        )
    for cp in out_copies:
        cp.wait()


def kernel(tokens, vocab_map):
    return _lookup(tokens.reshape(-1), vocab_map).reshape(tokens.shape)


# speedup vs baseline: 1.1174x; 1.0935x over previous
import functools

import jax
import jax.numpy as jnp
from jax import lax
from jax.experimental import pallas as pl
from jax.experimental.pallas import tpu as pltpu
from jax.experimental.pallas import tpu_sc as plsc

_VOCAB = 100000
_BATCH = 4096
_SEQ = 200
_NUM_CORES = 2
_NUM_SUBCORES = 16
_NW = _NUM_CORES * _NUM_SUBCORES
_ROWS_W = _BATCH // _NW
_PASS_ROWS = 64
_PASSES = _ROWS_W // _PASS_ROWS
_PASS_TOK = _PASS_ROWS * _SEQ
_LANES = 16

_mesh = plsc.VectorSubcoreMesh(core_axis_name="c", subcore_axis_name="s")


@functools.partial(
    pl.kernel,
    mesh=_mesh,
    out_type=jax.ShapeDtypeStruct((_BATCH, _SEQ), jnp.int32),
    scratch_types=[
        pltpu.VMEM((_VOCAB,), jnp.int32),
        pltpu.VMEM((_PASS_ROWS, _SEQ), jnp.int32),
        pltpu.SemaphoreType.DMA,
        pltpu.SemaphoreType.DMA,
    ],
    compiler_params=pltpu.CompilerParams(needs_layout_passes=False),
)
def _lookup(tokens_hbm, vocab_hbm, out_hbm, vocab_v, buf_v, sem_a, sem_b):
    wid = lax.axis_index("s") * _NUM_CORES + lax.axis_index("c")
    row0 = wid * _ROWS_W
    cp_vocab = pltpu.async_copy(vocab_hbm, vocab_v, sem_a)
    cp_vocab.wait()

    for p in range(_PASSES):
        r0 = row0 + p * _PASS_ROWS
        pltpu.sync_copy(tokens_hbm.at[pl.ds(r0, _PASS_ROWS)], buf_v)

        @plsc.parallel_loop(0, _PASS_TOK // _LANES, unroll=8)
        def _gather(i):
            pos = i * _LANES + jnp.arange(_LANES, dtype=jnp.int32)
            r = pos // _SEQ
            c = pos - r * _SEQ
            toks = plsc.load_gather(buf_v, [r, c])
            plsc.store_scatter(buf_v, [r, c], plsc.load_gather(vocab_v, [toks]))

        pltpu.sync_copy(buf_v, out_hbm.at[pl.ds(r0, _PASS_ROWS)])


def kernel(tokens, vocab_map):
    return _lookup(tokens, vocab_map)


# speedup vs baseline: 1.6291x; 1.4579x over previous
import functools

import jax
import jax.numpy as jnp
from jax import lax
from jax.experimental import pallas as pl
from jax.experimental.pallas import tpu as pltpu
from jax.experimental.pallas import tpu_sc as plsc

_VOCAB = 100000
_BATCH = 4096
_SEQ = 200
_NUM_CORES = 2
_NUM_SUBCORES = 16
_NW = _NUM_CORES * _NUM_SUBCORES
_COLS_W = _BATCH // _NW
_PER_W = _SEQ * _COLS_W
_LANES = 16

_mesh = plsc.VectorSubcoreMesh(core_axis_name="c", subcore_axis_name="s")


@functools.partial(
    pl.kernel,
    mesh=_mesh,
    out_type=jax.ShapeDtypeStruct((_SEQ, _BATCH), jnp.int32),
    scratch_types=[
        pltpu.VMEM((_VOCAB,), jnp.int32),
        pltpu.VMEM((_SEQ, _COLS_W), jnp.int32),
        pltpu.SemaphoreType.DMA,
        pltpu.SemaphoreType.DMA,
    ],
    compiler_params=pltpu.CompilerParams(needs_layout_passes=False),
)
def _lookup(tokens_hbm, vocab_hbm, out_hbm, vocab_v, buf_v, sem_a, sem_b):
    wid = lax.axis_index("s") * _NUM_CORES + lax.axis_index("c")
    col0 = wid * _COLS_W
    cp_vocab = pltpu.async_copy(vocab_hbm, vocab_v, sem_a)
    cp_tok = pltpu.async_copy(tokens_hbm.at[:, pl.ds(col0, _COLS_W)], buf_v, sem_b)
    cp_tok.wait()
    cp_vocab.wait()

    @plsc.parallel_loop(0, _PER_W // _LANES, unroll=8)
    def _gather(i):
        pos = i * _LANES + jnp.arange(_LANES, dtype=jnp.int32)
        r = pos >> 7
        c = pos & (_COLS_W - 1)
        toks = plsc.load_gather(buf_v, [r, c])
        plsc.store_scatter(buf_v, [r, c], plsc.load_gather(vocab_v, [toks]))

    pltpu.sync_copy(buf_v, out_hbm.at[:, pl.ds(col0, _COLS_W)])


def kernel(tokens, vocab_map):
    return _lookup(tokens.T, vocab_map).T


# speedup vs baseline: 1.9144x; 1.1751x over previous
import functools

import jax
import jax.numpy as jnp
from jax import lax
from jax.experimental import pallas as pl
from jax.experimental.pallas import tpu as pltpu
from jax.experimental.pallas import tpu_sc as plsc

_VOCAB = 100000
_BATCH = 4096
_SEQ = 200
_NUM_CORES = 2
_NUM_SUBCORES = 16
_NW = _NUM_CORES * _NUM_SUBCORES
_COLS_W = _BATCH // _NW
_PER_W = _SEQ * _COLS_W
_PASS1 = 104
_PASS2 = 96
_LANES = 16

_mesh = plsc.VectorSubcoreMesh(core_axis_name="c", subcore_axis_name="s")


@functools.partial(
    pl.kernel,
    mesh=_mesh,
    out_type=jax.ShapeDtypeStruct((_SEQ, _BATCH), jnp.int32),
    scratch_types=[
        pltpu.VMEM_SHARED((_VOCAB,), jnp.int32),
        pltpu.VMEM((_VOCAB,), jnp.int32),
        pltpu.VMEM((_PASS1, _COLS_W), jnp.int32),
        pltpu.SemaphoreType.DMA,
        pltpu.SemaphoreType.DMA,
    ],
    compiler_params=pltpu.CompilerParams(needs_layout_passes=False),
)
def _lookup(tokens_hbm, vocab_hbm, out_hbm, vocab_sh, vocab_v, buf_v, sem_a, sem_b):
    sid = lax.axis_index("s")
    wid = sid * _NUM_CORES + lax.axis_index("c")
    col0 = wid * _COLS_W
    cp_tok = pltpu.async_copy(
        tokens_hbm.at[pl.ds(0, _PASS1), pl.ds(col0, _COLS_W)], buf_v, sem_b
    )

    @pl.when(sid == 0)
    def _stage_vocab():
        pltpu.sync_copy(vocab_hbm, vocab_sh)

    plsc.subcore_barrier()
    cp_vocab = pltpu.async_copy(vocab_sh, vocab_v, sem_a)
    cp_vocab.wait()

    for p, (r0, rows) in enumerate([(0, _PASS1), (_PASS1, _PASS2)]):
        if p > 0:
            cp_tok = pltpu.async_copy(
                tokens_hbm.at[pl.ds(r0, rows), pl.ds(col0, _COLS_W)],
                buf_v.at[pl.ds(0, rows)],
                sem_b,
            )
        cp_tok.wait()

        @plsc.parallel_loop(0, rows * _COLS_W // _LANES, unroll=8)
        def _gather(i):
            pos = i * _LANES + jnp.arange(_LANES, dtype=jnp.int32)
            r = pos >> 7
            c = pos & (_COLS_W - 1)
            toks = plsc.load_gather(buf_v, [r, c])
            plsc.store_scatter(buf_v, [r, c], plsc.load_gather(vocab_v, [toks]))

        pltpu.sync_copy(
            buf_v.at[pl.ds(0, rows)],
            out_hbm.at[pl.ds(r0, rows), pl.ds(col0, _COLS_W)],
        )


def kernel(tokens, vocab_map):
    return _lookup(tokens.T, vocab_map).T
